# baseline (speedup 1.0000x reference)
import functools

import jax
import jax.numpy as jnp
from jax import lax
from jax.experimental import pallas as pl
from jax.experimental.pallas import tpu as pltpu
from jax.experimental.pallas import tpu_sc as plsc

MAX_LEN = 1000
NCOL = 26
D = 64
B = 16384

NC = 2
NS = 16
NW = NC * NS
L = 16

C = 416
NROWS = B * NCOL
NCHUNK = NROWS // C
IPT = NCHUNK // NW
GSUB = 4
GC = C // GSUB


def _mk_kernel():
    mesh = plsc.VectorSubcoreMesh(core_axis_name="c", subcore_axis_name="s")

    @functools.partial(
        pl.kernel,
        mesh=mesh,
        compiler_params=pltpu.CompilerParams(use_tc_tiling_on_sc=False),
        out_type=jax.ShapeDtypeStruct((NROWS, D), jnp.float32),
        scratch_types=[
            pltpu.VMEM((C,), jnp.int32),
            pltpu.VMEM((C,), jnp.int32),
            pltpu.VMEM((C,), jnp.int32),
            pltpu.VMEM((C, D), jnp.float32),
            pltpu.VMEM((C, D), jnp.float32),
            pltpu.SemaphoreType.DMA,
        ],
    )
    def k(ids_b_h, ids_s_h, bin_t_h, sub_t_h, pat_h, out_h,
          idx_b, idx_s, pat_v, rows_a, rows_b, sem):
        wid = lax.axis_index("s") * NC + lax.axis_index("c")
        pltpu.sync_copy(pat_h, pat_v)

        def item_body(t, _):
            g = wid * IPT + t
            r0 = g * C

            cpb = pltpu.async_copy(ids_b_h.at[pl.ds(r0, C)], idx_b, sem)
            cps = pltpu.async_copy(ids_s_h.at[pl.ds(r0, C)], idx_s, sem)
            cpb.wait()
            cps.wait()

            for kk in range(C // L):
                sl = pl.ds(kk * L, L)
                p = pat_v[sl]
                idx_b[sl] = idx_b[sl] + p
                idx_s[sl] = idx_s[sl] + p

            handles = []
            for j in range(GSUB):
                isl = pl.ds(j * GC, GC)
                handles.append(pltpu.async_copy(
                    bin_t_h.at[idx_b.at[isl]], rows_a.at[isl], sem))
                handles.append(pltpu.async_copy(
                    sub_t_h.at[idx_s.at[isl]], rows_b.at[isl], sem))
            for h in handles:
                h.wait()

            def add_body(r, _):
                row = r * 4
                for rr in range(4):
                    for k2 in range(D // L):
                        sl2 = pl.ds(k2 * L, L)
                        rows_a[row + rr, sl2] = (
                            rows_a[row + rr, sl2] + rows_b[row + rr, sl2])
                return _

            lax.fori_loop(0, C // 4, add_body, None)

            pltpu.sync_copy(rows_a, out_h.at[pl.ds(r0, C)])
            return _

        lax.fori_loop(0, IPT, item_body, None)

    return k


_sc_kernel = _mk_kernel()

_TCROWS = 128

_COLPERM = [(k % 2) * 13 + k // 2 for k in range(NCOL)]


def _tc_body(in_ref, out_ref):
    a = in_ref[...].reshape(_TCROWS, NCOL // 2, 2 * D)
    out_ref[...] = jnp.concatenate([a[:, :, :D], a[:, :, D:]], axis=1)


_tc_relayout = pl.pallas_call(
    _tc_body,
    grid=(B // _TCROWS,),
    in_specs=[pl.BlockSpec((_TCROWS * NCOL // 2, 2 * D), lambda g: (g, 0))],
    out_specs=pl.BlockSpec((_TCROWS, NCOL, D), lambda g: (g, 0, 0)),
    out_shape=jax.ShapeDtypeStruct((B, NCOL, D), jnp.float32),
)


def kernel(bin_ids, subbin_ids, bin_tables, subbin_tables):
    perm = jnp.asarray(_COLPERM, dtype=jnp.int32)
    ids_b = bin_ids.astype(jnp.int32)[:, perm].reshape(-1)
    ids_s = subbin_ids.astype(jnp.int32)[:, perm].reshape(-1)
    bin_t = bin_tables.reshape(NCOL * MAX_LEN, D)
    sub_t = subbin_tables.reshape(NCOL * MAX_LEN, D)
    pat = jnp.tile(jnp.asarray(_COLPERM, dtype=jnp.int32) * MAX_LEN,
                   C // NCOL)
    out = _sc_kernel(ids_b, ids_s, bin_t, sub_t, pat)
    return _tc_relayout(out.reshape(NROWS // 2, 2 * D))


# speedup vs baseline: 1.5931x; 1.5931x over previous
import functools

import jax
import jax.numpy as jnp
from jax import lax
from jax.experimental import pallas as pl
from jax.experimental.pallas import tpu as pltpu
from jax.experimental.pallas import tpu_sc as plsc

MAX_LEN = 1000
NCOL = 26
D = 64
B = 16384

NC = 2
NS = 16
NW = NC * NS
L = 16

NDO = D // 8
NHALF = 2
HB = B // NHALF
ITEMS = NCOL * NDO * NHALF
IPT = ITEMS // NW
NCHUNK = HB // 512
NBT = B // 128


def _mk_kernel():
    mesh = plsc.VectorSubcoreMesh(core_axis_name="c", subcore_axis_name="s")

    @functools.partial(
        pl.kernel,
        mesh=mesh,
        compiler_params=pltpu.CompilerParams(
            use_tc_tiling_on_sc=False, needs_layout_passes=False),
        out_type=jax.ShapeDtypeStruct((NCOL, 8, NBT, 1024), jnp.float32),
        scratch_types=[
            pltpu.VMEM((8 * MAX_LEN,), jnp.float32),
            pltpu.VMEM((8 * MAX_LEN,), jnp.float32),
            pltpu.VMEM((HB,), jnp.int32),
            pltpu.VMEM((HB,), jnp.int32),
            pltpu.VMEM((4, 1024), jnp.float32),
            pltpu.VMEM((4, 1024), jnp.float32),
            pltpu.SemaphoreType.DMA,
            pltpu.SemaphoreType.DMA,
            pltpu.SemaphoreType.DMA,
        ],
    )
    def k(ids_b_h, ids_s_h, bin_h, sub_h, out_h,
          bslab, sslab, idv_b, idv_s, stg_a, stg_b, isem, osem_a, osem_b):
        wid = lax.axis_index("s") * NC + lax.axis_index("c")

        def item_body(i, _):
            g = wid * IPT + i
            c = g // (NDO * NHALF)
            rem = g % (NDO * NHALF)
            do = rem // NHALF
            half = rem % NHALF

            row0 = c * D + do * 8
            b_off = c * B + half * HB
            cps = [
                pltpu.async_copy(
                    bin_h.at[pl.ds(row0 * MAX_LEN, 8 * MAX_LEN)], bslab, isem),
                pltpu.async_copy(
                    sub_h.at[pl.ds(row0 * MAX_LEN, 8 * MAX_LEN)], sslab, isem),
                pltpu.async_copy(ids_b_h.at[pl.ds(b_off, HB)], idv_b, isem),
                pltpu.async_copy(ids_s_h.at[pl.ds(b_off, HB)], idv_s, isem),
            ]
            for cp in cps:
                cp.wait()

            bt_base = half * (HB // 128)

            def chunk(ch, stg, osem, first):
                @pl.when(jnp.logical_not(first))
                def _w():
                    pltpu.make_async_copy(
                        stg, out_h.at[0, 0, pl.ds(0, 4)], osem).wait()

                def bcg_body(q, _):
                    base = ch * 512 + q * L
                    ivb = idv_b[pl.ds(base, L)]
                    ivs = idv_s[pl.ds(base, L)]
                    bt = q // 8
                    lane0 = (q % 8) * L
                    for dr in range(8):
                        va = plsc.load_gather(bslab, [ivb + dr * MAX_LEN])
                        vs = plsc.load_gather(sslab, [ivs + dr * MAX_LEN])
                        stg[bt, pl.ds(dr * 128 + lane0, L)] = va + vs
                    return _

                lax.fori_loop(0, 32, bcg_body, None)
                pltpu.async_copy(
                    stg, out_h.at[c, do, pl.ds(bt_base + ch * 4, 4)], osem)

            def chunk_pair(p, _):
                chunk(p * 2, stg_a, osem_a, jnp.logical_and(i == 0, p == 0))
                chunk(p * 2 + 1, stg_b, osem_b,
                      jnp.logical_and(i == 0, p == 0))
                return _

            lax.fori_loop(0, NCHUNK // 2, chunk_pair, None)
            return _

        lax.fori_loop(0, IPT, item_body, None)
        pltpu.make_async_copy(stg_a, out_h.at[0, 0, pl.ds(0, 4)], osem_a).wait()
        pltpu.make_async_copy(stg_b, out_h.at[0, 0, pl.ds(0, 4)], osem_b).wait()

    return k


_sc_kernel = _mk_kernel()


def kernel(bin_ids, subbin_ids, bin_tables, subbin_tables):
    ids_b = jnp.transpose(bin_ids.astype(jnp.int32)).reshape(-1)
    ids_s = jnp.transpose(subbin_ids.astype(jnp.int32)).reshape(-1)
    bin_t = jnp.transpose(bin_tables, (0, 2, 1)).reshape(-1)
    sub_t = jnp.transpose(subbin_tables, (0, 2, 1)).reshape(-1)
    out4 = _sc_kernel(ids_b, ids_s, bin_t, sub_t)
    out = (out4.reshape(NCOL, 8, NBT, 8, 128)
           .transpose(0, 1, 3, 2, 4)
           .reshape(NCOL, D, B)
           .transpose(2, 0, 1))
    return out


# speedup vs baseline: 3.3086x; 2.0769x over previous
import functools

import jax
import jax.numpy as jnp
from jax import lax
from jax.experimental import pallas as pl
from jax.experimental.pallas import tpu as pltpu
from jax.experimental.pallas import tpu_sc as plsc

MAX_LEN = 1000
NCOL = 26
D = 64
B = 16384

NC = 2
NS = 16
NW = NC * NS
L = 16

NDO = D // 8
NHALF = 2
HB = B // NHALF
ITEMS = NCOL * NDO * NHALF
IPT = ITEMS // NW
NCHUNK = HB // 512
NBT = B // 128


def _mk_kernel():
    mesh = plsc.VectorSubcoreMesh(core_axis_name="c", subcore_axis_name="s")

    @functools.partial(
        pl.kernel,
        mesh=mesh,
        compiler_params=pltpu.CompilerParams(
            use_tc_tiling_on_sc=False, needs_layout_passes=False),
        out_type=jax.ShapeDtypeStruct((NCOL, 8, NBT, 1024), jnp.float32),
        scratch_types=[
            pltpu.VMEM((8 * MAX_LEN,), jnp.float32),
            pltpu.VMEM((8 * MAX_LEN,), jnp.float32),
            pltpu.VMEM((HB,), jnp.int32),
            pltpu.VMEM((HB,), jnp.int32),
            pltpu.VMEM((4, 1024), jnp.float32),
            pltpu.VMEM((4, 1024), jnp.float32),
            pltpu.SemaphoreType.DMA,
            pltpu.SemaphoreType.DMA,
            pltpu.SemaphoreType.DMA,
        ],
    )
    def k(ids_b_h, ids_s_h, bin_h, sub_h, out_h,
          bslab, sslab, idv_b, idv_s, stg_a, stg_b, isem, osem_a, osem_b):
        wid = lax.axis_index("s") * NC + lax.axis_index("c")

        def item_body(i, _):
            g = wid * IPT + i
            c = g // (NDO * NHALF)
            rem = g % (NDO * NHALF)
            half = rem // NDO
            do = rem % NDO

            row0 = c * D + do * 8
            b_off = c * B + half * HB
            cps = [
                pltpu.async_copy(
                    bin_h.at[pl.ds(row0 * MAX_LEN, 8 * MAX_LEN)], bslab, isem),
                pltpu.async_copy(
                    sub_h.at[pl.ds(row0 * MAX_LEN, 8 * MAX_LEN)], sslab, isem),
            ]

            @pl.when(jnp.logical_or(i == 0, do == 0))
            def _ids():
                cpi = pltpu.async_copy(
                    ids_b_h.at[pl.ds(b_off, HB)], idv_b, isem)
                cps2 = pltpu.async_copy(
                    ids_s_h.at[pl.ds(b_off, HB)], idv_s, isem)
                cpi.wait()
                cps2.wait()

            for cp in cps:
                cp.wait()

            bt_base = half * (HB // 128)

            def chunk(ch, stg, osem, first):
                @pl.when(jnp.logical_not(first))
                def _w():
                    pltpu.make_async_copy(
                        stg, out_h.at[0, 0, pl.ds(0, 4)], osem).wait()

                def bcg_body(q, _):
                    base = ch * 512 + q * L
                    ivb = idv_b[pl.ds(base, L)]
                    ivs = idv_s[pl.ds(base, L)]
                    bt = q // 8
                    lane0 = (q % 8) * L
                    vas = [plsc.load_gather(bslab, [ivb + dr * MAX_LEN])
                           for dr in range(8)]
                    vss = [plsc.load_gather(sslab, [ivs + dr * MAX_LEN])
                           for dr in range(8)]
                    sums = [vas[dr] + vss[dr] for dr in range(8)]
                    for dr in range(8):
                        stg[bt, pl.ds(dr * 128 + lane0, L)] = sums[dr]
                    return _

                lax.fori_loop(0, 32, bcg_body, None)
                pltpu.async_copy(
                    stg, out_h.at[c, do, pl.ds(bt_base + ch * 4, 4)], osem)

            def chunk_pair(p, _):
                chunk(p * 2, stg_a, osem_a, jnp.logical_and(i == 0, p == 0))
                chunk(p * 2 + 1, stg_b, osem_b,
                      jnp.logical_and(i == 0, p == 0))
                return _

            lax.fori_loop(0, NCHUNK // 2, chunk_pair, None)
            return _

        lax.fori_loop(0, IPT, item_body, None)
        pltpu.make_async_copy(stg_a, out_h.at[0, 0, pl.ds(0, 4)], osem_a).wait()
        pltpu.make_async_copy(stg_b, out_h.at[0, 0, pl.ds(0, 4)], osem_b).wait()

    return k


_sc_kernel = _mk_kernel()


def kernel(bin_ids, subbin_ids, bin_tables, subbin_tables):
    ids_b = jnp.transpose(bin_ids.astype(jnp.int32)).reshape(-1)
    ids_s = jnp.transpose(subbin_ids.astype(jnp.int32)).reshape(-1)
    bin_t = jnp.transpose(bin_tables, (0, 2, 1)).reshape(-1)
    sub_t = jnp.transpose(subbin_tables, (0, 2, 1)).reshape(-1)
    out4 = _sc_kernel(ids_b, ids_s, bin_t, sub_t)
    out = (out4.reshape(NCOL, 8, NBT, 8, 128)
           .transpose(0, 1, 3, 2, 4)
           .reshape(NCOL, D, B)
           .transpose(2, 0, 1))
    return out


# speedup vs baseline: 3.7249x; 1.1258x over previous
import functools

import jax
import jax.numpy as jnp
from jax import lax
from jax.experimental import pallas as pl
from jax.experimental.pallas import tpu as pltpu
from jax.experimental.pallas import tpu_sc as plsc

MAX_LEN = 1000
NCOL = 26
D = 64
B = 16384

NC = 2
NS = 16
NW = NC * NS
L = 16

NDO = D // 8
NHALF = 2
HB = B // NHALF
ITEMS = NCOL * NDO * NHALF
IPT = ITEMS // NW
NCHUNK = HB // 512
NBT = B // 128


def _mk_kernel():
    mesh = plsc.VectorSubcoreMesh(core_axis_name="c", subcore_axis_name="s")

    @functools.partial(
        pl.kernel,
        mesh=mesh,
        compiler_params=pltpu.CompilerParams(
            use_tc_tiling_on_sc=False, needs_layout_passes=False),
        out_type=jax.ShapeDtypeStruct((NCOL, 8, NBT, 1024), jnp.float32),
        scratch_types=[
            pltpu.VMEM((8 * MAX_LEN,), jnp.float32),
            pltpu.VMEM((8 * MAX_LEN,), jnp.float32),
            pltpu.VMEM((8 * MAX_LEN,), jnp.float32),
            pltpu.VMEM((8 * MAX_LEN,), jnp.float32),
            pltpu.VMEM((HB,), jnp.int32),
            pltpu.VMEM((HB,), jnp.int32),
            pltpu.VMEM((4, 1024), jnp.float32),
            pltpu.VMEM((4, 1024), jnp.float32),
            pltpu.SemaphoreType.DMA,
            pltpu.SemaphoreType.DMA,
            pltpu.SemaphoreType.DMA,
            pltpu.SemaphoreType.DMA,
            pltpu.SemaphoreType.DMA,
        ],
    )
    def k(ids_b_h, ids_s_h, bin_h, sub_h, out_h,
          bsl_a, ssl_a, bsl_b, ssl_b, idv_b, idv_s, stg_a, stg_b,
          isem, ssem_a, ssem_b, osem_a, osem_b):
        wid = lax.axis_index("s") * NC + lax.axis_index("c")

        def item_coords(i):
            g = wid * IPT + i
            c = g // (NDO * NHALF)
            rem = g % (NDO * NHALF)
            half = rem // NDO
            do = rem % NDO
            return c, half, do

        def slab_start(i, bsl, ssl, ssem):
            c, _, do = item_coords(i)
            row0 = c * D + do * 8
            pltpu.async_copy(
                bin_h.at[pl.ds(row0 * MAX_LEN, 8 * MAX_LEN)], bsl, ssem)
            pltpu.async_copy(
                sub_h.at[pl.ds(row0 * MAX_LEN, 8 * MAX_LEN)], ssl, ssem)

        def slab_wait(bsl, ssl, ssem):
            pltpu.make_async_copy(
                bin_h.at[pl.ds(0, 8 * MAX_LEN)], bsl, ssem).wait()
            pltpu.make_async_copy(
                sub_h.at[pl.ds(0, 8 * MAX_LEN)], ssl, ssem).wait()

        def item_body(i, bslab, sslab, ssem, nbsl, nssl, nsem):
            c, half, do = item_coords(i)
            b_off = c * B + half * HB

            @pl.when(jnp.logical_or(i == 0, do == 0))
            def _ids():
                cpi = pltpu.async_copy(
                    ids_b_h.at[pl.ds(b_off, HB)], idv_b, isem)
                cps2 = pltpu.async_copy(
                    ids_s_h.at[pl.ds(b_off, HB)], idv_s, isem)
                cpi.wait()
                cps2.wait()

            slab_wait(bslab, sslab, ssem)

            @pl.when(i + 1 < IPT)
            def _pref():
                slab_start(i + 1, nbsl, nssl, nsem)

            bt_base = half * (HB // 128)

            def chunk(ch, stg, osem, first):
                @pl.when(jnp.logical_not(first))
                def _w():
                    pltpu.make_async_copy(
                        stg, out_h.at[0, 0, pl.ds(0, 4)], osem).wait()

                def one_q(q):
                    base = ch * 512 + q * L
                    ivb = idv_b[pl.ds(base, L)]
                    ivs = idv_s[pl.ds(base, L)]
                    bt = q // 8
                    lane0 = (q % 8) * L
                    vas = [plsc.load_gather(
                        bslab.at[pl.ds(dr * MAX_LEN, MAX_LEN)], [ivb])
                        for dr in range(8)]
                    vss = [plsc.load_gather(
                        sslab.at[pl.ds(dr * MAX_LEN, MAX_LEN)], [ivs])
                        for dr in range(8)]
                    sums = [vas[dr] + vss[dr] for dr in range(8)]
                    for dr in range(8):
                        stg[bt, pl.ds(dr * 128 + lane0, L)] = sums[dr]

                def bcg_body(u, _):
                    one_q(u * 2)
                    one_q(u * 2 + 1)
                    return _

                lax.fori_loop(0, 16, bcg_body, None)
                pltpu.async_copy(
                    stg, out_h.at[c, do, pl.ds(bt_base + ch * 4, 4)], osem)

            def chunk_pair(p, _):
                chunk(p * 2, stg_a, osem_a, jnp.logical_and(i == 0, p == 0))
                chunk(p * 2 + 1, stg_b, osem_b,
                      jnp.logical_and(i == 0, p == 0))
                return _

            lax.fori_loop(0, NCHUNK // 2, chunk_pair, None)

        slab_start(0, bsl_a, ssl_a, ssem_a)

        def item_pair(p, _):
            item_body(p * 2, bsl_a, ssl_a, ssem_a, bsl_b, ssl_b, ssem_b)
            item_body(p * 2 + 1, bsl_b, ssl_b, ssem_b, bsl_a, ssl_a, ssem_a)
            return _

        lax.fori_loop(0, IPT // 2, item_pair, None)
        item_body(IPT - 1, bsl_a, ssl_a, ssem_a, bsl_b, ssl_b, ssem_b)
        pltpu.make_async_copy(stg_a, out_h.at[0, 0, pl.ds(0, 4)], osem_a).wait()
        pltpu.make_async_copy(stg_b, out_h.at[0, 0, pl.ds(0, 4)], osem_b).wait()

    return k


_sc_kernel = _mk_kernel()


def kernel(bin_ids, subbin_ids, bin_tables, subbin_tables):
    ids_b = jnp.transpose(bin_ids.astype(jnp.int32)).reshape(-1)
    ids_s = jnp.transpose(subbin_ids.astype(jnp.int32)).reshape(-1)
    bin_t = jnp.transpose(bin_tables, (0, 2, 1)).reshape(-1)
    sub_t = jnp.transpose(subbin_tables, (0, 2, 1)).reshape(-1)
    out4 = _sc_kernel(ids_b, ids_s, bin_t, sub_t)
    out = (out4.reshape(NCOL, 8, NBT, 8, 128)
           .transpose(0, 1, 3, 2, 4)
           .reshape(NCOL, D, B)
           .transpose(2, 0, 1))
    return out
